# Initial kernel scaffold; baseline (speedup 1.0000x reference)
import jax
import jax.numpy as jnp
from jax.experimental import pallas as pl


def kernel(x, rdf_feat, bdf_feat, atom_idx, ele_idx, W_rdf, b_rdf, g_rdf, be_rdf, W_bdf, b_bdf, g_bdf, be_bdf, Wf, bf):
    raise NotImplementedError("write your pallas kernel here")


# speedup vs baseline: 2.6853x; 2.6853x over previous
import functools

import jax
import jax.numpy as jnp
from jax import lax
from jax.experimental import pallas as pl
from jax.experimental.pallas import tpu as pltpu
import jax.experimental.pallas.tpu_sc as plsc

F32 = jnp.float32

_NUM_ATOMS = 10000
_NUM_ELE = 100
_N = 640000

_NC = 2
_NS = 16
_NW = _NC * _NS
_ROWS_W = _N // _NW
_SC_CHUNK = 1000
_NCHUNK = _ROWS_W // _SC_CHUNK

_BLK = 2000
_GRID = _N // _BLK

def _sc_mesh():
    return plsc.VectorSubcoreMesh(
        core_axis_name="c", subcore_axis_name="s",
        num_cores=_NC, num_subcores=_NS,
    )


def _sc_segsum(x, atom_idx, ele_idx):
    zA = jnp.zeros((_NUM_ATOMS, 64), F32)
    zA1 = jnp.zeros((_NUM_ATOMS, 8), F32)
    zE = jnp.zeros((_NUM_ELE, 64), F32)
    zE1 = jnp.zeros((_NUM_ELE, 8), F32)
    ones = jnp.ones((_SC_CHUNK, 8), F32)

    @functools.partial(
        pl.kernel,
        out_type=(
            jax.ShapeDtypeStruct((_NC, _NUM_ATOMS, 64), F32),
            jax.ShapeDtypeStruct((_NC, _NUM_ATOMS, 8), F32),
            jax.ShapeDtypeStruct((_NC, _NUM_ELE, 64), F32),
            jax.ShapeDtypeStruct((_NC, _NUM_ELE, 8), F32),
        ),
        mesh=_sc_mesh(),
        compiler_params=pltpu.CompilerParams(use_tc_tiling_on_sc=False),
        scratch_types=[
            pltpu.VMEM((_SC_CHUNK, 64), F32),
            pltpu.VMEM((_SC_CHUNK,), jnp.int32),
            pltpu.VMEM((_SC_CHUNK,), jnp.int32),
            pltpu.VMEM((_SC_CHUNK, 8), F32),
            pltpu.VMEM_SHARED((_NUM_ATOMS, 64), F32),
            pltpu.VMEM_SHARED((_NUM_ATOMS, 8), F32),
            pltpu.VMEM_SHARED((_NUM_ELE, 64), F32),
            pltpu.VMEM_SHARED((_NUM_ELE, 8), F32),
        ],
    )
    def k(x_hbm, ai_hbm, ei_hbm, zA_hbm, zA1_hbm, zE_hbm, zE1_hbm, ones_hbm,
          saP, caP, seP, ceP,
          xbuf, aibuf, eibuf, onesbuf, sa_sh, ca_sh, se_sh, ce_sh):
        cid = lax.axis_index("c")
        sid = lax.axis_index("s")
        wid = cid * _NS + sid

        @pl.when(sid == 0)
        def _():
            pltpu.sync_copy(zA_hbm, sa_sh)
            pltpu.sync_copy(zA1_hbm, ca_sh)
            pltpu.sync_copy(zE_hbm, se_sh)
            pltpu.sync_copy(zE1_hbm, ce_sh)

        pltpu.sync_copy(ones_hbm, onesbuf)
        plsc.subcore_barrier()

        def step(i, carry):
            base = wid * _ROWS_W + i * _SC_CHUNK
            pltpu.sync_copy(x_hbm.at[pl.ds(base, _SC_CHUNK)], xbuf)
            pltpu.sync_copy(ai_hbm.at[pl.ds(base, _SC_CHUNK)], aibuf)
            pltpu.sync_copy(ei_hbm.at[pl.ds(base, _SC_CHUNK)], eibuf)
            pltpu.sync_copy(xbuf, sa_sh.at[aibuf], add=True)
            pltpu.sync_copy(onesbuf, ca_sh.at[aibuf], add=True)
            pltpu.sync_copy(xbuf, se_sh.at[eibuf], add=True)
            pltpu.sync_copy(onesbuf, ce_sh.at[eibuf], add=True)
            return carry

        lax.fori_loop(0, _NCHUNK, step, 0)
        plsc.subcore_barrier()

        @pl.when(sid == 0)
        def _():
            pltpu.sync_copy(sa_sh, saP.at[cid])
            pltpu.sync_copy(ca_sh, caP.at[cid])
            pltpu.sync_copy(se_sh, seP.at[cid])
            pltpu.sync_copy(ce_sh, ceP.at[cid])

    return k(x, atom_idx, ele_idx, zA, zA1, zE, zE1, ones)


def _tc_tables(saP, caP, seP, ceP):
    def body(saP_ref, caP_ref, seP_ref, ceP_ref, A_ref, E_ref):
        sa = saP_ref[0, :, :32] + saP_ref[1, :, :32]
        ca = jnp.maximum(caP_ref[0, :, 0:1] + caP_ref[1, :, 0:1], 1.0)
        A_ref[...] = jnp.maximum(sa / ca, 0.0)
        se = seP_ref[0, :, 32:] + seP_ref[1, :, 32:]
        ce = jnp.maximum(ceP_ref[0, :, 0:1] + ceP_ref[1, :, 0:1], 1.0)
        E_ref[...] = jnp.maximum(se / ce, 0.0)

    return pl.pallas_call(
        body,
        out_shape=(
            jax.ShapeDtypeStruct((_NUM_ATOMS, 32), F32),
            jax.ShapeDtypeStruct((_NUM_ELE, 32), F32),
        ),
    )(saP, caP, seP, ceP)


def _sc_gather(A, E, atom_idx, ele_idx):
    @functools.partial(
        pl.kernel,
        out_type=(
            jax.ShapeDtypeStruct((_N, 32), F32),
            jax.ShapeDtypeStruct((_N, 32), F32),
        ),
        mesh=_sc_mesh(),
        compiler_params=pltpu.CompilerParams(use_tc_tiling_on_sc=False),
        scratch_types=[
            pltpu.VMEM((_SC_CHUNK,), jnp.int32),
            pltpu.VMEM((_SC_CHUNK,), jnp.int32),
            pltpu.VMEM((_SC_CHUNK, 32), F32),
            pltpu.VMEM((_SC_CHUNK, 32), F32),
            pltpu.SemaphoreType.DMA,
            pltpu.SemaphoreType.DMA,
        ],
    )
    def k(A_hbm, E_hbm, ai_hbm, ei_hbm, Ag_hbm, Eg_hbm,
          aibuf, eibuf, abuf, ebuf, sema, seme):
        cid = lax.axis_index("c")
        sid = lax.axis_index("s")
        wid = cid * _NS + sid

        def step(i, carry):
            base = wid * _ROWS_W + i * _SC_CHUNK
            pltpu.sync_copy(ai_hbm.at[pl.ds(base, _SC_CHUNK)], aibuf)
            pltpu.sync_copy(ei_hbm.at[pl.ds(base, _SC_CHUNK)], eibuf)
            da = pltpu.async_copy(A_hbm.at[aibuf], abuf, sema)
            de = pltpu.async_copy(E_hbm.at[eibuf], ebuf, seme)
            da.wait()
            de.wait()
            pltpu.sync_copy(abuf, Ag_hbm.at[pl.ds(base, _SC_CHUNK)])
            pltpu.sync_copy(ebuf, Eg_hbm.at[pl.ds(base, _SC_CHUNK)])
            return carry

        lax.fori_loop(0, _NCHUNK, step, 0)

    return k(A, E, atom_idx, ele_idx)


def _tc_stats(rdf, bdf, ag, eg, wr, wb):
    def body(rdf_ref, bdf_ref, ag_ref, eg_ref, wr_ref, wb_ref, acc_ref):
        a = ag_ref[...]
        e = eg_ref[...]
        zr = jnp.concatenate([rdf_ref[...], a, e], axis=1)
        hr = jnp.dot(zr, wr_ref[...], preferred_element_type=F32)
        zb = jnp.concatenate([bdf_ref[...], a, e], axis=1)
        hb = jnp.dot(zb, wb_ref[...], preferred_element_type=F32)
        vals = jnp.concatenate(
            [
                jnp.sum(hr, axis=0, keepdims=True),
                jnp.sum(hr * hr, axis=0, keepdims=True),
                jnp.sum(hb, axis=0, keepdims=True),
                jnp.sum(hb * hb, axis=0, keepdims=True),
            ],
            axis=0,
        )

        @pl.when(pl.program_id(0) == 0)
        def _():
            acc_ref[...] = jnp.zeros_like(acc_ref)

        acc_ref[0:4, :] += vals

    return pl.pallas_call(
        body,
        grid=(_GRID,),
        in_specs=[pl.BlockSpec((_BLK, 32), lambda i: (i, 0))] * 4
        + [pl.BlockSpec((96, 64), lambda i: (0, 0))] * 2,
        out_specs=pl.BlockSpec((8, 64), lambda i: (0, 0)),
        out_shape=jax.ShapeDtypeStruct((8, 64), F32),
    )(rdf, bdf, ag, eg, wr, wb)


def _tc_apply(x, rdf, bdf, ag, eg, wr, wb, P, wf1, wf2):
    def body(x_ref, rdf_ref, bdf_ref, ag_ref, eg_ref, wr_ref, wb_ref,
             P_ref, wf1_ref, wf2_ref, out_ref):
        a = ag_ref[...]
        e = eg_ref[...]
        xv = x_ref[...]
        zr = jnp.concatenate([rdf_ref[...], a, e], axis=1)
        hr = jnp.dot(zr, wr_ref[...], preferred_element_type=F32)
        x1 = jnp.maximum(hr * P_ref[0:1, :] + P_ref[1:2, :] + xv, 0.0)
        zb = jnp.concatenate([bdf_ref[...], a, e], axis=1)
        hb = jnp.dot(zb, wb_ref[...], preferred_element_type=F32)
        x2 = jnp.maximum(hb * P_ref[2:3, :] + P_ref[3:4, :] + xv, 0.0)
        o = (
            jnp.dot(x1, wf1_ref[...], preferred_element_type=F32)
            + jnp.dot(x2, wf2_ref[...], preferred_element_type=F32)
            + P_ref[4:5, :]
        )
        out_ref[...] = jnp.maximum(o, 0.0)

    return pl.pallas_call(
        body,
        grid=(_GRID,),
        in_specs=[pl.BlockSpec((_BLK, 64), lambda i: (i, 0))]
        + [pl.BlockSpec((_BLK, 32), lambda i: (i, 0))] * 4
        + [pl.BlockSpec((96, 64), lambda i: (0, 0))] * 2
        + [pl.BlockSpec((8, 64), lambda i: (0, 0))]
        + [pl.BlockSpec((64, 64), lambda i: (0, 0))] * 2,
        out_specs=pl.BlockSpec((_BLK, 64), lambda i: (i, 0)),
        out_shape=jax.ShapeDtypeStruct((_N, 64), F32),
    )(x, rdf, bdf, ag, eg, wr, wb, P, wf1, wf2)


def kernel(x, rdf_feat, bdf_feat, atom_idx, ele_idx, W_rdf, b_rdf, g_rdf,
           be_rdf, W_bdf, b_bdf, g_bdf, be_bdf, Wf, bf):
    del b_rdf, b_bdf
    saP, caP, seP, ceP = _sc_segsum(x, atom_idx, ele_idx)
    A, E = _tc_tables(saP, caP, seP, ceP)
    ag, eg = _sc_gather(A, E, atom_idx, ele_idx)

    wr = W_rdf.T
    wb = W_bdf.T
    acc = _tc_stats(rdf_feat, bdf_feat, ag, eg, wr, wb)

    n = jnp.float32(_N)
    mu_r = acc[0] / n
    var_r = jnp.maximum(acc[1] / n - mu_r * mu_r, 0.0)
    sr = g_rdf * lax.rsqrt(var_r + 1e-5)
    tr = be_rdf - mu_r * sr
    mu_b = acc[2] / n
    var_b = jnp.maximum(acc[3] / n - mu_b * mu_b, 0.0)
    sb = g_bdf * lax.rsqrt(var_b + 1e-5)
    tb = be_bdf - mu_b * sb
    P = jnp.concatenate(
        [
            sr[None], tr[None], sb[None], tb[None], bf[None],
            jnp.zeros((3, 64), F32),
        ],
        axis=0,
    )

    return _tc_apply(x, rdf_feat, bdf_feat, ag, eg, wr, wb, P,
                     Wf[:, :64].T, Wf[:, 64:].T)


# speedup vs baseline: 2.8721x; 1.0696x over previous
import functools

import jax
import jax.numpy as jnp
from jax import lax
from jax.experimental import pallas as pl
from jax.experimental.pallas import tpu as pltpu
import jax.experimental.pallas.tpu_sc as plsc

F32 = jnp.float32

_NUM_ATOMS = 10000
_NUM_ELE = 100
_N = 640000

_NC = 2
_NS = 16
_NW = _NC * _NS
_ROWS_W = _N // _NW
_SC_CHUNK = 1000
_NCHUNK = _ROWS_W // _SC_CHUNK

_BLK = 2000
_GRID = _N // _BLK

def _sc_mesh():
    return plsc.VectorSubcoreMesh(
        core_axis_name="c", subcore_axis_name="s",
        num_cores=_NC, num_subcores=_NS,
    )


_ELE_PAD = 128


def _sc_segsum(x, atom_idx, ele_idx):
    zA = jnp.zeros((_NUM_ATOMS, 32), F32)
    zA1 = jnp.zeros((_NUM_ATOMS, 8), F32)
    zE = jnp.zeros((_ELE_PAD, 32), F32)
    zE1 = jnp.zeros((_ELE_PAD, 8), F32)
    ones = jnp.ones((_SC_CHUNK, 8), F32)

    @functools.partial(
        pl.kernel,
        out_type=(
            jax.ShapeDtypeStruct((_NC, _NUM_ATOMS, 32), F32),
            jax.ShapeDtypeStruct((_NC, _NUM_ATOMS, 8), F32),
            jax.ShapeDtypeStruct((_NC, _ELE_PAD, 32), F32),
            jax.ShapeDtypeStruct((_NC, _ELE_PAD, 8), F32),
        ),
        mesh=_sc_mesh(),
        compiler_params=pltpu.CompilerParams(use_tc_tiling_on_sc=False),
        scratch_types=[
            pltpu.VMEM((_SC_CHUNK, 32), F32),
            pltpu.VMEM((_SC_CHUNK, 32), F32),
            pltpu.VMEM((_SC_CHUNK,), jnp.int32),
            pltpu.VMEM((_SC_CHUNK,), jnp.int32),
            pltpu.VMEM((_SC_CHUNK, 8), F32),
            pltpu.VMEM_SHARED((_NUM_ATOMS, 32), F32),
            pltpu.VMEM_SHARED((_NUM_ATOMS, 8), F32),
            pltpu.VMEM_SHARED((_ELE_PAD, 32), F32),
            pltpu.VMEM_SHARED((_ELE_PAD, 8), F32),
        ],
    )
    def k(x_hbm, ai_hbm, ei_hbm, zA_hbm, zA1_hbm, zE_hbm, zE1_hbm, ones_hbm,
          saP, caP, seP, ceP,
          xbufA, xbufE, aibuf, eibuf, onesbuf, sa_sh, ca_sh, se_sh, ce_sh):
        cid = lax.axis_index("c")
        sid = lax.axis_index("s")
        wid = cid * _NS + sid

        @pl.when(sid == 0)
        def _():
            pltpu.sync_copy(zA_hbm, sa_sh)
            pltpu.sync_copy(zA1_hbm, ca_sh)
            pltpu.sync_copy(zE_hbm, se_sh)
            pltpu.sync_copy(zE1_hbm, ce_sh)

        pltpu.sync_copy(ones_hbm, onesbuf)
        plsc.subcore_barrier()

        def step(i, carry):
            base = wid * _ROWS_W + i * _SC_CHUNK
            pltpu.sync_copy(x_hbm.at[pl.ds(base, _SC_CHUNK), pl.ds(0, 32)],
                            xbufA)
            pltpu.sync_copy(x_hbm.at[pl.ds(base, _SC_CHUNK), pl.ds(32, 32)],
                            xbufE)
            pltpu.sync_copy(ai_hbm.at[pl.ds(base, _SC_CHUNK)], aibuf)
            pltpu.sync_copy(ei_hbm.at[pl.ds(base, _SC_CHUNK)], eibuf)
            pltpu.sync_copy(xbufA, sa_sh.at[aibuf], add=True)
            pltpu.sync_copy(onesbuf, ca_sh.at[aibuf], add=True)
            pltpu.sync_copy(xbufE, se_sh.at[eibuf], add=True)
            pltpu.sync_copy(onesbuf, ce_sh.at[eibuf], add=True)
            return carry

        lax.fori_loop(0, _NCHUNK, step, 0)
        plsc.subcore_barrier()

        @pl.when(sid == 0)
        def _():
            pltpu.sync_copy(sa_sh, saP.at[cid])
            pltpu.sync_copy(ca_sh, caP.at[cid])
            pltpu.sync_copy(se_sh, seP.at[cid])
            pltpu.sync_copy(ce_sh, ceP.at[cid])

    return k(x, atom_idx, ele_idx, zA, zA1, zE, zE1, ones)


def _tc_tables(saP, caP, seP, ceP, wr, wb):
    def body(saP_ref, caP_ref, seP_ref, ceP_ref, wr_ref, wb_ref,
             A_ref, wrp_ref, wbp_ref):
        sa = saP_ref[0] + saP_ref[1]
        ca = jnp.maximum(caP_ref[0, :, 0:1] + caP_ref[1, :, 0:1], 1.0)
        A_ref[...] = jnp.maximum(sa / ca, 0.0)
        se = seP_ref[0] + seP_ref[1]
        ce = jnp.maximum(ceP_ref[0, :, 0:1] + ceP_ref[1, :, 0:1], 1.0)
        E = jnp.maximum(se / ce, 0.0)
        wrp_ref[0:64, :] = wr_ref[0:64, :]
        wrp_ref[64:, :] = jnp.dot(E, wr_ref[64:, :],
                                  preferred_element_type=F32)
        wbp_ref[0:64, :] = wb_ref[0:64, :]
        wbp_ref[64:, :] = jnp.dot(E, wb_ref[64:, :],
                                  preferred_element_type=F32)

    return pl.pallas_call(
        body,
        out_shape=(
            jax.ShapeDtypeStruct((_NUM_ATOMS, 32), F32),
            jax.ShapeDtypeStruct((64 + _ELE_PAD, 64), F32),
            jax.ShapeDtypeStruct((64 + _ELE_PAD, 64), F32),
        ),
    )(saP, caP, seP, ceP, wr, wb)


def _sc_gather(A, atom_idx):
    @functools.partial(
        pl.kernel,
        out_type=jax.ShapeDtypeStruct((_N, 32), F32),
        mesh=_sc_mesh(),
        compiler_params=pltpu.CompilerParams(use_tc_tiling_on_sc=False),
        scratch_types=[
            pltpu.VMEM((_SC_CHUNK,), jnp.int32),
            pltpu.VMEM((_SC_CHUNK, 32), F32),
            pltpu.SemaphoreType.DMA,
        ],
    )
    def k(A_hbm, ai_hbm, Ag_hbm, aibuf, abuf, sema):
        cid = lax.axis_index("c")
        sid = lax.axis_index("s")
        wid = cid * _NS + sid

        def step(i, carry):
            base = wid * _ROWS_W + i * _SC_CHUNK
            pltpu.sync_copy(ai_hbm.at[pl.ds(base, _SC_CHUNK)], aibuf)
            da = pltpu.async_copy(A_hbm.at[aibuf], abuf, sema)
            da.wait()
            pltpu.sync_copy(abuf, Ag_hbm.at[pl.ds(base, _SC_CHUNK)])
            return carry

        lax.fori_loop(0, _NCHUNK, step, 0)

    return k(A, atom_idx)


def _onehot(ei):
    return (ei == lax.broadcasted_iota(jnp.int32, (_BLK, _ELE_PAD), 1)
            ).astype(F32)


def _tc_stats(rdf, bdf, ag, ei, wrp, wbp):
    def body(rdf_ref, bdf_ref, ag_ref, ei_ref, wrp_ref, wbp_ref, acc_ref):
        a = ag_ref[...]
        oh = _onehot(ei_ref[...])
        zr = jnp.concatenate([rdf_ref[...], a, oh], axis=1)
        hr = jnp.dot(zr, wrp_ref[...], preferred_element_type=F32)
        zb = jnp.concatenate([bdf_ref[...], a, oh], axis=1)
        hb = jnp.dot(zb, wbp_ref[...], preferred_element_type=F32)
        vals = jnp.concatenate(
            [
                jnp.sum(hr, axis=0, keepdims=True),
                jnp.sum(hr * hr, axis=0, keepdims=True),
                jnp.sum(hb, axis=0, keepdims=True),
                jnp.sum(hb * hb, axis=0, keepdims=True),
            ],
            axis=0,
        )

        @pl.when(pl.program_id(0) == 0)
        def _():
            acc_ref[...] = jnp.zeros_like(acc_ref)

        acc_ref[0:4, :] += vals

    return pl.pallas_call(
        body,
        grid=(_GRID,),
        in_specs=[pl.BlockSpec((_BLK, 32), lambda i: (i, 0))] * 3
        + [pl.BlockSpec((_BLK, 1), lambda i: (i, 0))]
        + [pl.BlockSpec((64 + _ELE_PAD, 64), lambda i: (0, 0))] * 2,
        out_specs=pl.BlockSpec((8, 64), lambda i: (0, 0)),
        out_shape=jax.ShapeDtypeStruct((8, 64), F32),
    )(rdf, bdf, ag, ei, wrp, wbp)


def _tc_apply(x, rdf, bdf, ag, ei, wrp, wbp, P, wf1, wf2):
    def body(x_ref, rdf_ref, bdf_ref, ag_ref, ei_ref, wrp_ref, wbp_ref,
             P_ref, wf1_ref, wf2_ref, out_ref):
        a = ag_ref[...]
        oh = _onehot(ei_ref[...])
        xv = x_ref[...]
        zr = jnp.concatenate([rdf_ref[...], a, oh], axis=1)
        hr = jnp.dot(zr, wrp_ref[...], preferred_element_type=F32)
        x1 = jnp.maximum(hr * P_ref[0:1, :] + P_ref[1:2, :] + xv, 0.0)
        zb = jnp.concatenate([bdf_ref[...], a, oh], axis=1)
        hb = jnp.dot(zb, wbp_ref[...], preferred_element_type=F32)
        x2 = jnp.maximum(hb * P_ref[2:3, :] + P_ref[3:4, :] + xv, 0.0)
        o = (
            jnp.dot(x1, wf1_ref[...], preferred_element_type=F32)
            + jnp.dot(x2, wf2_ref[...], preferred_element_type=F32)
            + P_ref[4:5, :]
        )
        out_ref[...] = jnp.maximum(o, 0.0)

    return pl.pallas_call(
        body,
        grid=(_GRID,),
        in_specs=[pl.BlockSpec((_BLK, 64), lambda i: (i, 0))]
        + [pl.BlockSpec((_BLK, 32), lambda i: (i, 0))] * 3
        + [pl.BlockSpec((_BLK, 1), lambda i: (i, 0))]
        + [pl.BlockSpec((64 + _ELE_PAD, 64), lambda i: (0, 0))] * 2
        + [pl.BlockSpec((8, 64), lambda i: (0, 0))]
        + [pl.BlockSpec((64, 64), lambda i: (0, 0))] * 2,
        out_specs=pl.BlockSpec((_BLK, 64), lambda i: (i, 0)),
        out_shape=jax.ShapeDtypeStruct((_N, 64), F32),
    )(x, rdf, bdf, ag, ei, wrp, wbp, P, wf1, wf2)


def kernel(x, rdf_feat, bdf_feat, atom_idx, ele_idx, W_rdf, b_rdf, g_rdf,
           be_rdf, W_bdf, b_bdf, g_bdf, be_bdf, Wf, bf):
    del b_rdf, b_bdf
    saP, caP, seP, ceP = _sc_segsum(x, atom_idx, ele_idx)
    A, wrp, wbp = _tc_tables(saP, caP, seP, ceP, W_rdf.T, W_bdf.T)
    ag = _sc_gather(A, atom_idx)

    ei2 = ele_idx.reshape(_N, 1)
    acc = _tc_stats(rdf_feat, bdf_feat, ag, ei2, wrp, wbp)

    n = jnp.float32(_N)
    mu_r = acc[0] / n
    var_r = jnp.maximum(acc[1] / n - mu_r * mu_r, 0.0)
    sr = g_rdf * lax.rsqrt(var_r + 1e-5)
    tr = be_rdf - mu_r * sr
    mu_b = acc[2] / n
    var_b = jnp.maximum(acc[3] / n - mu_b * mu_b, 0.0)
    sb = g_bdf * lax.rsqrt(var_b + 1e-5)
    tb = be_bdf - mu_b * sb
    P = jnp.concatenate(
        [
            sr[None], tr[None], sb[None], tb[None], bf[None],
            jnp.zeros((3, 64), F32),
        ],
        axis=0,
    )

    return _tc_apply(x, rdf_feat, bdf_feat, ag, ei2, wrp, wbp, P,
                     Wf[:, :64].T, Wf[:, 64:].T)


# speedup vs baseline: 3.1805x; 1.1074x over previous
import functools

import jax
import jax.numpy as jnp
from jax import lax
from jax.experimental import pallas as pl
from jax.experimental.pallas import tpu as pltpu
import jax.experimental.pallas.tpu_sc as plsc

F32 = jnp.float32

_NUM_ATOMS = 10000
_NUM_ELE = 100
_N = 640000

_NC = 2
_NS = 16
_NW = _NC * _NS
_ROWS_W = _N // _NW
_SC_CHUNK = 1000
_NCHUNK = _ROWS_W // _SC_CHUNK

_BLK = 8000
_GRID = _N // _BLK

def _sc_mesh():
    return plsc.VectorSubcoreMesh(
        core_axis_name="c", subcore_axis_name="s",
        num_cores=_NC, num_subcores=_NS,
    )


_ELE_PAD = 128


def _sc_segsum(x, atom_idx, ele_idx):
    zA = jnp.zeros((_NUM_ATOMS, 32), F32)
    zA1 = jnp.zeros((_NUM_ATOMS, 8), F32)
    zE = jnp.zeros((_ELE_PAD, 32), F32)
    zE1 = jnp.zeros((_ELE_PAD, 8), F32)
    ones = jnp.ones((_SC_CHUNK, 8), F32)

    @functools.partial(
        pl.kernel,
        out_type=(
            jax.ShapeDtypeStruct((_NC, _NUM_ATOMS, 32), F32),
            jax.ShapeDtypeStruct((_NC, _NUM_ATOMS, 8), F32),
            jax.ShapeDtypeStruct((_NC, _ELE_PAD, 32), F32),
            jax.ShapeDtypeStruct((_NC, _ELE_PAD, 8), F32),
        ),
        mesh=_sc_mesh(),
        compiler_params=pltpu.CompilerParams(use_tc_tiling_on_sc=False),
        scratch_types=[
            pltpu.VMEM((_SC_CHUNK, 32), F32),
            pltpu.VMEM((_SC_CHUNK, 32), F32),
            pltpu.VMEM((_SC_CHUNK,), jnp.int32),
            pltpu.VMEM((_SC_CHUNK,), jnp.int32),
            pltpu.VMEM((_SC_CHUNK, 8), F32),
            pltpu.VMEM_SHARED((_NUM_ATOMS, 32), F32),
            pltpu.VMEM_SHARED((_NUM_ATOMS, 8), F32),
            pltpu.VMEM_SHARED((_ELE_PAD, 32), F32),
            pltpu.VMEM_SHARED((_ELE_PAD, 8), F32),
        ],
    )
    def k(x_hbm, ai_hbm, ei_hbm, zA_hbm, zA1_hbm, zE_hbm, zE1_hbm, ones_hbm,
          saP, caP, seP, ceP,
          xbufA, xbufE, aibuf, eibuf, onesbuf, sa_sh, ca_sh, se_sh, ce_sh):
        cid = lax.axis_index("c")
        sid = lax.axis_index("s")
        wid = cid * _NS + sid

        @pl.when(sid == 0)
        def _():
            pltpu.sync_copy(zA_hbm, sa_sh)
            pltpu.sync_copy(zA1_hbm, ca_sh)
            pltpu.sync_copy(zE_hbm, se_sh)
            pltpu.sync_copy(zE1_hbm, ce_sh)

        pltpu.sync_copy(ones_hbm, onesbuf)
        plsc.subcore_barrier()

        def step(i, carry):
            base = wid * _ROWS_W + i * _SC_CHUNK
            pltpu.sync_copy(x_hbm.at[pl.ds(base, _SC_CHUNK), pl.ds(0, 32)],
                            xbufA)
            pltpu.sync_copy(x_hbm.at[pl.ds(base, _SC_CHUNK), pl.ds(32, 32)],
                            xbufE)
            pltpu.sync_copy(ai_hbm.at[pl.ds(base, _SC_CHUNK)], aibuf)
            pltpu.sync_copy(ei_hbm.at[pl.ds(base, _SC_CHUNK)], eibuf)
            pltpu.sync_copy(xbufA, sa_sh.at[aibuf], add=True)
            pltpu.sync_copy(onesbuf, ca_sh.at[aibuf], add=True)
            pltpu.sync_copy(xbufE, se_sh.at[eibuf], add=True)
            pltpu.sync_copy(onesbuf, ce_sh.at[eibuf], add=True)
            return carry

        lax.fori_loop(0, _NCHUNK, step, 0)
        plsc.subcore_barrier()

        @pl.when(sid == 0)
        def _():
            pltpu.sync_copy(sa_sh, saP.at[cid])
            pltpu.sync_copy(ca_sh, caP.at[cid])
            pltpu.sync_copy(se_sh, seP.at[cid])
            pltpu.sync_copy(ce_sh, ceP.at[cid])

    return k(x, atom_idx, ele_idx, zA, zA1, zE, zE1, ones)


def _tc_tables(saP, caP, seP, ceP, wr, wb):
    def body(saP_ref, caP_ref, seP_ref, ceP_ref, wr_ref, wb_ref,
             A_ref, wrp_ref, wbp_ref):
        sa = saP_ref[0] + saP_ref[1]
        ca = jnp.maximum(caP_ref[0, :, 0:1] + caP_ref[1, :, 0:1], 1.0)
        A_ref[...] = jnp.maximum(sa / ca, 0.0)
        se = seP_ref[0] + seP_ref[1]
        ce = jnp.maximum(ceP_ref[0, :, 0:1] + ceP_ref[1, :, 0:1], 1.0)
        E = jnp.maximum(se / ce, 0.0)
        wrp_ref[0:64, :] = wr_ref[0:64, :]
        wrp_ref[64:, :] = jnp.dot(E, wr_ref[64:, :],
                                  preferred_element_type=F32)
        wbp_ref[0:64, :] = wb_ref[0:64, :]
        wbp_ref[64:, :] = jnp.dot(E, wb_ref[64:, :],
                                  preferred_element_type=F32)

    return pl.pallas_call(
        body,
        out_shape=(
            jax.ShapeDtypeStruct((_NUM_ATOMS, 32), F32),
            jax.ShapeDtypeStruct((64 + _ELE_PAD, 64), F32),
            jax.ShapeDtypeStruct((64 + _ELE_PAD, 64), F32),
        ),
    )(saP, caP, seP, ceP, wr, wb)


def _sc_gather(A, atom_idx):
    @functools.partial(
        pl.kernel,
        out_type=jax.ShapeDtypeStruct((_N, 32), F32),
        mesh=_sc_mesh(),
        compiler_params=pltpu.CompilerParams(use_tc_tiling_on_sc=False),
        scratch_types=[
            pltpu.VMEM((_SC_CHUNK,), jnp.int32),
            pltpu.VMEM((_SC_CHUNK, 32), F32),
            pltpu.SemaphoreType.DMA,
        ],
    )
    def k(A_hbm, ai_hbm, Ag_hbm, aibuf, abuf, sema):
        cid = lax.axis_index("c")
        sid = lax.axis_index("s")
        wid = cid * _NS + sid

        def step(i, carry):
            base = wid * _ROWS_W + i * _SC_CHUNK
            pltpu.sync_copy(ai_hbm.at[pl.ds(base, _SC_CHUNK)], aibuf)
            da = pltpu.async_copy(A_hbm.at[aibuf], abuf, sema)
            da.wait()
            pltpu.sync_copy(abuf, Ag_hbm.at[pl.ds(base, _SC_CHUNK)])
            return carry

        lax.fori_loop(0, _NCHUNK, step, 0)

    return k(A, atom_idx)


def _onehot(ei):
    return (ei == lax.broadcasted_iota(jnp.int32, (_BLK, _ELE_PAD), 1)
            ).astype(F32)


def _tc_stats(rdf, bdf, ag, ei, wrp, wbp):
    def body(rdf_ref, bdf_ref, ag_ref, ei_ref, wrp_ref, wbp_ref, acc_ref):
        a = ag_ref[...]
        oh = _onehot(ei_ref[...])
        zr = jnp.concatenate([rdf_ref[...], a, oh], axis=1)
        hr = jnp.dot(zr, wrp_ref[...], preferred_element_type=F32)
        zb = jnp.concatenate([bdf_ref[...], a, oh], axis=1)
        hb = jnp.dot(zb, wbp_ref[...], preferred_element_type=F32)
        vals = jnp.concatenate(
            [
                jnp.sum(hr, axis=0, keepdims=True),
                jnp.sum(hr * hr, axis=0, keepdims=True),
                jnp.sum(hb, axis=0, keepdims=True),
                jnp.sum(hb * hb, axis=0, keepdims=True),
            ],
            axis=0,
        )

        @pl.when(pl.program_id(0) == 0)
        def _():
            acc_ref[...] = jnp.zeros_like(acc_ref)

        acc_ref[0:4, :] += vals

    return pl.pallas_call(
        body,
        grid=(_GRID,),
        in_specs=[pl.BlockSpec((_BLK, 32), lambda i: (i, 0))] * 3
        + [pl.BlockSpec((_BLK, 1), lambda i: (i, 0))]
        + [pl.BlockSpec((64 + _ELE_PAD, 64), lambda i: (0, 0))] * 2,
        out_specs=pl.BlockSpec((8, 64), lambda i: (0, 0)),
        out_shape=jax.ShapeDtypeStruct((8, 64), F32),
    )(rdf, bdf, ag, ei, wrp, wbp)


def _tc_apply(x, rdf, bdf, ag, ei, wrp, wbp, P, wf1, wf2):
    def body(x_ref, rdf_ref, bdf_ref, ag_ref, ei_ref, wrp_ref, wbp_ref,
             P_ref, wf1_ref, wf2_ref, out_ref):
        a = ag_ref[...]
        oh = _onehot(ei_ref[...])
        xv = x_ref[...]
        zr = jnp.concatenate([rdf_ref[...], a, oh], axis=1)
        hr = jnp.dot(zr, wrp_ref[...], preferred_element_type=F32)
        x1 = jnp.maximum(hr * P_ref[0:1, :] + P_ref[1:2, :] + xv, 0.0)
        zb = jnp.concatenate([bdf_ref[...], a, oh], axis=1)
        hb = jnp.dot(zb, wbp_ref[...], preferred_element_type=F32)
        x2 = jnp.maximum(hb * P_ref[2:3, :] + P_ref[3:4, :] + xv, 0.0)
        o = (
            jnp.dot(x1, wf1_ref[...], preferred_element_type=F32)
            + jnp.dot(x2, wf2_ref[...], preferred_element_type=F32)
            + P_ref[4:5, :]
        )
        out_ref[...] = jnp.maximum(o, 0.0)

    return pl.pallas_call(
        body,
        grid=(_GRID,),
        in_specs=[pl.BlockSpec((_BLK, 64), lambda i: (i, 0))]
        + [pl.BlockSpec((_BLK, 32), lambda i: (i, 0))] * 3
        + [pl.BlockSpec((_BLK, 1), lambda i: (i, 0))]
        + [pl.BlockSpec((64 + _ELE_PAD, 64), lambda i: (0, 0))] * 2
        + [pl.BlockSpec((8, 64), lambda i: (0, 0))]
        + [pl.BlockSpec((64, 64), lambda i: (0, 0))] * 2,
        out_specs=pl.BlockSpec((_BLK, 64), lambda i: (i, 0)),
        out_shape=jax.ShapeDtypeStruct((_N, 64), F32),
    )(x, rdf, bdf, ag, ei, wrp, wbp, P, wf1, wf2)


def kernel(x, rdf_feat, bdf_feat, atom_idx, ele_idx, W_rdf, b_rdf, g_rdf,
           be_rdf, W_bdf, b_bdf, g_bdf, be_bdf, Wf, bf):
    del b_rdf, b_bdf
    saP, caP, seP, ceP = _sc_segsum(x, atom_idx, ele_idx)
    A, wrp, wbp = _tc_tables(saP, caP, seP, ceP, W_rdf.T, W_bdf.T)
    ag = _sc_gather(A, atom_idx)

    ei2 = ele_idx.reshape(_N, 1)
    acc = _tc_stats(rdf_feat, bdf_feat, ag, ei2, wrp, wbp)

    n = jnp.float32(_N)
    mu_r = acc[0] / n
    var_r = jnp.maximum(acc[1] / n - mu_r * mu_r, 0.0)
    sr = g_rdf * lax.rsqrt(var_r + 1e-5)
    tr = be_rdf - mu_r * sr
    mu_b = acc[2] / n
    var_b = jnp.maximum(acc[3] / n - mu_b * mu_b, 0.0)
    sb = g_bdf * lax.rsqrt(var_b + 1e-5)
    tb = be_bdf - mu_b * sb
    P = jnp.concatenate(
        [
            sr[None], tr[None], sb[None], tb[None], bf[None],
            jnp.zeros((3, 64), F32),
        ],
        axis=0,
    )

    return _tc_apply(x, rdf_feat, bdf_feat, ag, ei2, wrp, wbp, P,
                     Wf[:, :64].T, Wf[:, 64:].T)
